# Initial kernel scaffold; baseline (speedup 1.0000x reference)
import jax
import jax.numpy as jnp
from jax.experimental import pallas as pl


def kernel(x, edge_index, W1, b1, W2, b2, W3, b3, Wih0, Whh0, bih0, bhh0, Wih1, Whh1, bih1, bhh1, Wih2, Whh2, bih2, bhh2, Wlin, blin):
    raise NotImplementedError("write your pallas kernel here")


# speedup vs baseline: 6.9472x; 6.9472x over previous
import functools

import jax
import jax.numpy as jnp
from jax import lax
from jax.experimental import pallas as pl
from jax.experimental.pallas import tpu as pltpu
from jax.experimental.pallas import tpu_sc as plsc

N = 10000
D = 128
HD = D // 2
E = 320000
NTILES = 16
NCORES = 2
NW = NTILES * NCORES
CHUNK = 128
NCH = 80
EP = NW * NCH * CHUNK
NP = 10240
RPT = NP // NTILES

_mesh = plsc.VectorSubcoreMesh(core_axis_name="c", subcore_axis_name="s")


@functools.partial(
    pl.kernel,
    out_type=jax.ShapeDtypeStruct((NCORES, NP), jnp.float32),
    mesh=_mesh,
    scratch_types=[
        pltpu.VMEM((NCH, CHUNK), jnp.int32),
        pltpu.VMEM((CHUNK,), jnp.float32),
        pltpu.VMEM((RPT,), jnp.float32),
        pltpu.VMEM_SHARED((NP,), jnp.float32),
    ],
)
def _sc_degree(dst_hbm, ones_hbm, zeros_hbm, deg_out, dst_v, ones_v, zer_v, acc):
    cid = lax.axis_index("c")
    sid = lax.axis_index("s")
    wid = sid * NCORES + cid
    pltpu.sync_copy(dst_hbm.at[wid], dst_v)
    pltpu.sync_copy(ones_hbm, ones_v)
    pltpu.sync_copy(zeros_hbm, zer_v)
    pltpu.sync_copy(zer_v, acc.at[pl.ds(sid * RPT, RPT)])
    plsc.subcore_barrier()

    @pl.loop(0, NCH)
    def _(j):
        pltpu.sync_copy(ones_v, acc.at[dst_v.at[j]], add=True)

    plsc.subcore_barrier()
    pltpu.sync_copy(acc.at[pl.ds(sid * RPT, RPT)],
                    deg_out.at[cid, pl.ds(sid * RPT, RPT)])


@functools.partial(
    pl.kernel,
    out_type=jax.ShapeDtypeStruct((NCORES, 2, NP, HD), jnp.float32),
    mesh=_mesh,
    compiler_params=pltpu.CompilerParams(use_tc_tiling_on_sc=False),
    scratch_types=[
        pltpu.VMEM((NCH, CHUNK), jnp.int32),
        pltpu.VMEM((NCH, CHUNK), jnp.int32),
        pltpu.VMEM((CHUNK, HD), jnp.float32),
        pltpu.VMEM((CHUNK, HD), jnp.float32),
        pltpu.VMEM_SHARED((NP, HD), jnp.float32),
        pltpu.SemaphoreType.DMA,
        pltpu.SemaphoreType.DMA,
    ],
)
def _sc_spmm(t_hbm, src_hbm, dst_hbm, zeros_hbm, out,
             src_v, dst_v, buf0, buf1, acc, sem0, sem1):
    cid = lax.axis_index("c")
    sid = lax.axis_index("s")
    wid = sid * NCORES + cid
    pltpu.sync_copy(src_hbm.at[wid], src_v)
    pltpu.sync_copy(dst_hbm.at[wid], dst_v)

    bufs = (buf0, buf1)
    sems = (sem0, sem1)
    for h in range(2):
        th = t_hbm.at[h]
        pltpu.sync_copy(zeros_hbm, acc.at[pl.ds(sid * RPT, RPT)])
        plsc.subcore_barrier()
        pltpu.async_copy(th.at[src_v.at[0]], bufs[0], sems[0])

        @pl.loop(0, NCH, step=2)
        def _(j):
            for b in range(2):
                jb = j + b
                nxt = jb + 1
                pltpu.make_async_copy(th.at[src_v.at[0]], bufs[b],
                                      sems[b]).wait()

                @pl.when(nxt < NCH)
                def _():
                    pltpu.async_copy(th.at[src_v.at[nxt]], bufs[1 - b],
                                     sems[1 - b])

                pltpu.sync_copy(bufs[b], acc.at[dst_v.at[jb]], add=True)

        plsc.subcore_barrier()
        pltpu.sync_copy(acc.at[pl.ds(sid * RPT, RPT)],
                        out.at[cid, h, pl.ds(sid * RPT, RPT)])


def _dot_t(a, w):
    return lax.dot_general(a, w, (((1,), (1,)), ((), ())),
                           preferred_element_type=jnp.float32)


def _split_store(t_ref, t_full):
    t_ref[0] = t_full[:, :HD]
    t_ref[1] = t_full[:, HD:]


def _merge(p_ref, t_ref):
    p = jnp.concatenate([p_ref[0, 0] + p_ref[1, 0],
                         p_ref[0, 1] + p_ref[1, 1]], axis=1)
    t = jnp.concatenate([t_ref[0], t_ref[1]], axis=1)
    return p + t


def _tc_pre_kernel(deg_ref, x_ref, w1_ref, dinv_ref, t1_ref):
    deg = deg_ref[0] + deg_ref[1] + 1.0
    row = lax.broadcasted_iota(jnp.int32, (NP, 1), 0)
    dinv = jnp.where(row < N, lax.rsqrt(deg)[:, None], 0.0)
    dinv_ref[...] = dinv
    _split_store(t1_ref, _dot_t(x_ref[...], w1_ref[...]) * dinv)


def _tc_mid_kernel(p_ref, t_ref, dinv_ref, b_ref, w_ref, tn_ref):
    dinv = dinv_ref[...]
    z = jax.nn.relu(dinv * _merge(p_ref, t_ref) + b_ref[...])
    _split_store(tn_ref, _dot_t(z, w_ref[...]) * dinv)


def _tc_post_kernel(p_ref, t_ref, dinv_ref, b3_ref,
                    wih0_ref, bl0_ref, wih1_ref, bl1_ref, wih2_ref, bl2_ref,
                    wlin_ref, blin_ref, out_ref):
    dinv = dinv_ref[...]
    h = jax.nn.relu(dinv * _merge(p_ref, t_ref) + b3_ref[...])
    for wih_ref, bl_ref in ((wih0_ref, bl0_ref), (wih1_ref, bl1_ref),
                            (wih2_ref, bl2_ref)):
        gates = _dot_t(h, wih_ref[...]) + bl_ref[...]
        i = jax.nn.sigmoid(gates[:, 0 * D:1 * D])
        g = jnp.tanh(gates[:, 2 * D:3 * D])
        o = jax.nn.sigmoid(gates[:, 3 * D:4 * D])
        h = o * jnp.tanh(i * g)
    out_ref[...] = _dot_t(h, wlin_ref[...]) + blin_ref[...]


def _tc_call(body, out_shapes, *args):
    return pl.pallas_call(body, out_shape=out_shapes)(*args)


def kernel(x, edge_index, W1, b1, W2, b2, W3, b3,
           Wih0, Whh0, bih0, bhh0, Wih1, Whh1, bih1, bhh1,
           Wih2, Whh2, bih2, bhh2, Wlin, blin):
    ei = edge_index.astype(jnp.int32)
    pad = jnp.full((EP - E,), N, jnp.int32)
    src = jnp.concatenate([ei[0], pad]).reshape(NW, NCH, CHUNK)
    dst = jnp.concatenate([ei[1], pad]).reshape(NW, NCH, CHUNK)

    x_pad = jnp.concatenate([x, jnp.zeros((NP - N, D), x.dtype)])
    ones_c = jnp.ones((CHUNK,), jnp.float32)
    zeros_r = jnp.zeros((RPT,), jnp.float32)
    zeros_rh = jnp.zeros((RPT, HD), jnp.float32)

    deg = _sc_degree(dst, ones_c, zeros_r)

    t_shape = jax.ShapeDtypeStruct((2, NP, HD), jnp.float32)
    dinv, t1 = _tc_call(
        _tc_pre_kernel,
        (jax.ShapeDtypeStruct((NP, 1), jnp.float32), t_shape),
        deg, x_pad, W1)

    p1 = _sc_spmm(t1, src, dst, zeros_rh)
    t2 = _tc_call(_tc_mid_kernel, t_shape,
                  p1, t1, dinv, b1.reshape(1, D), W2)

    p2 = _sc_spmm(t2, src, dst, zeros_rh)
    t3 = _tc_call(_tc_mid_kernel, t_shape,
                  p2, t2, dinv, b2.reshape(1, D), W3)

    p3 = _sc_spmm(t3, src, dst, zeros_rh)
    out = _tc_call(
        _tc_post_kernel, jax.ShapeDtypeStruct((NP, D), jnp.float32),
        p3, t3, dinv, b3.reshape(1, D),
        Wih0, (bih0 + bhh0).reshape(1, 4 * D),
        Wih1, (bih1 + bhh1).reshape(1, 4 * D),
        Wih2, (bih2 + bhh2).reshape(1, 4 * D),
        Wlin, blin.reshape(1, D))

    return out[:N]


# speedup vs baseline: 15.1877x; 2.1862x over previous
import functools

import jax
import jax.numpy as jnp
from jax import lax
from jax.experimental import pallas as pl
from jax.experimental.pallas import tpu as pltpu
from jax.experimental.pallas import tpu_sc as plsc

N = 10000
D = 128
NPASS = 4
PW = D // NPASS
E = 320000
NTILES = 16
NCORES = 2
NW = NTILES * NCORES
CHUNK = 128
NCH = 80
EP = NW * NCH * CHUNK
NP = 10240
RPT = NP // NTILES

_mesh = plsc.VectorSubcoreMesh(core_axis_name="c", subcore_axis_name="s")


@functools.partial(
    pl.kernel,
    out_type=jax.ShapeDtypeStruct((NCORES, NP), jnp.float32),
    mesh=_mesh,
    scratch_types=[
        pltpu.VMEM((NCH, CHUNK), jnp.int32),
        pltpu.VMEM((CHUNK,), jnp.float32),
        pltpu.VMEM((RPT,), jnp.float32),
        pltpu.VMEM_SHARED((NP,), jnp.float32),
    ],
)
def _sc_degree(dst_hbm, ones_hbm, zeros_hbm, deg_out, dst_v, ones_v, zer_v, acc):
    cid = lax.axis_index("c")
    sid = lax.axis_index("s")
    wid = sid * NCORES + cid
    pltpu.sync_copy(dst_hbm.at[wid], dst_v)
    pltpu.sync_copy(ones_hbm, ones_v)
    pltpu.sync_copy(zeros_hbm, zer_v)
    pltpu.sync_copy(zer_v, acc.at[pl.ds(sid * RPT, RPT)])
    plsc.subcore_barrier()

    @pl.loop(0, NCH)
    def _(j):
        pltpu.sync_copy(ones_v, acc.at[dst_v.at[j]], add=True)

    plsc.subcore_barrier()
    pltpu.sync_copy(acc.at[pl.ds(sid * RPT, RPT)],
                    deg_out.at[cid, pl.ds(sid * RPT, RPT)])


@functools.partial(
    pl.kernel,
    out_type=jax.ShapeDtypeStruct((NCORES, NPASS, NP, PW), jnp.float32),
    mesh=_mesh,
    compiler_params=pltpu.CompilerParams(use_tc_tiling_on_sc=False),
    scratch_types=[
        pltpu.VMEM((NCH, CHUNK), jnp.int32),
        pltpu.VMEM((NCH, CHUNK), jnp.int32),
        pltpu.VMEM((CHUNK, PW), jnp.float32),
        pltpu.VMEM((CHUNK, PW), jnp.float32),
        pltpu.VMEM_SHARED((NP, PW), jnp.float32),
        pltpu.VMEM_SHARED((NP, PW), jnp.float32),
        pltpu.SemaphoreType.DMA,
        pltpu.SemaphoreType.DMA,
    ],
)
def _sc_spmm(t_hbm, src_hbm, dst_hbm, zeros_hbm, out,
             src_v, dst_v, buf0, buf1, acc, tsp, sem0, sem1):
    cid = lax.axis_index("c")
    sid = lax.axis_index("s")
    wid = sid * NCORES + cid
    pltpu.sync_copy(src_hbm.at[wid], src_v)
    pltpu.sync_copy(dst_hbm.at[wid], dst_v)

    bufs = (buf0, buf1)
    sems = (sem0, sem1)
    slab = pl.ds(sid * RPT, RPT)
    for h in range(NPASS):
        pltpu.sync_copy(t_hbm.at[h, slab], tsp.at[slab])
        pltpu.sync_copy(zeros_hbm, acc.at[slab])
        plsc.subcore_barrier()
        pltpu.async_copy(tsp.at[src_v.at[0]], bufs[0], sems[0])

        @pl.loop(0, NCH, step=2)
        def _(j):
            for b in range(2):
                jb = j + b
                nxt = jb + 1
                pltpu.make_async_copy(tsp.at[src_v.at[0]], bufs[b],
                                      sems[b]).wait()

                @pl.when(nxt < NCH)
                def _():
                    pltpu.async_copy(tsp.at[src_v.at[nxt]], bufs[1 - b],
                                     sems[1 - b])

                pltpu.sync_copy(bufs[b], acc.at[dst_v.at[jb]], add=True)

        plsc.subcore_barrier()
        pltpu.sync_copy(acc.at[slab], out.at[cid, h, slab])


def _dot_t(a, w):
    return lax.dot_general(a, w, (((1,), (1,)), ((), ())),
                           preferred_element_type=jnp.float32)


def _split_store(t_ref, t_full):
    for h in range(NPASS):
        t_ref[h] = t_full[:, h * PW:(h + 1) * PW]


def _merge(p_ref, t_ref):
    p = jnp.concatenate([p_ref[0, h] + p_ref[1, h] for h in range(NPASS)],
                        axis=1)
    t = jnp.concatenate([t_ref[h] for h in range(NPASS)], axis=1)
    return p + t


def _tc_pre_kernel(deg_ref, x_ref, w1_ref, dinv_ref, t1_ref):
    deg = deg_ref[0] + deg_ref[1] + 1.0
    row = pl.program_id(0) * BR + lax.broadcasted_iota(jnp.int32, (BR, 1), 0)
    dinv = jnp.where(row < N, lax.rsqrt(deg)[:, None], 0.0)
    dinv_ref[...] = dinv
    _split_store(t1_ref, _dot_t(x_ref[...], w1_ref[...]) * dinv)


def _tc_mid_kernel(p_ref, t_ref, dinv_ref, b_ref, w_ref, tn_ref):
    dinv = dinv_ref[...]
    z = jax.nn.relu(dinv * _merge(p_ref, t_ref) + b_ref[...])
    _split_store(tn_ref, _dot_t(z, w_ref[...]) * dinv)


def _tc_post_kernel(p_ref, t_ref, dinv_ref, b3_ref,
                    wih0_ref, bl0_ref, wih1_ref, bl1_ref, wih2_ref, bl2_ref,
                    wlin_ref, blin_ref, out_ref):
    dinv = dinv_ref[...]
    h = jax.nn.relu(dinv * _merge(p_ref, t_ref) + b3_ref[...])
    for wih_ref, bl_ref in ((wih0_ref, bl0_ref), (wih1_ref, bl1_ref),
                            (wih2_ref, bl2_ref)):
        gates = _dot_t(h, wih_ref[...]) + bl_ref[...]
        i = jax.nn.sigmoid(gates[:, 0 * D:1 * D])
        g = jnp.tanh(gates[:, 2 * D:3 * D])
        o = jax.nn.sigmoid(gates[:, 3 * D:4 * D])
        h = o * jnp.tanh(i * g)
    out_ref[...] = _dot_t(h, wlin_ref[...]) + blin_ref[...]


BR = 2048

_spec_rows = pl.BlockSpec((BR, D), lambda i: (i, 0))
_spec_rows1 = pl.BlockSpec((BR, 1), lambda i: (i, 0))
_spec_deg = pl.BlockSpec((NCORES, BR), lambda i: (0, i))
_spec_t = pl.BlockSpec((NPASS, BR, PW), lambda i: (0, i, 0))
_spec_p = pl.BlockSpec((NCORES, NPASS, BR, PW), lambda i: (0, 0, i, 0))


def _spec_full(shape):
    return pl.BlockSpec(shape, lambda i: tuple(0 for _ in shape))


def _tc_call(body, in_specs, out_specs, out_shapes, *args):
    return pl.pallas_call(
        body,
        grid=(NP // BR,),
        in_specs=in_specs,
        out_specs=out_specs,
        out_shape=out_shapes,
    )(*args)


def kernel(x, edge_index, W1, b1, W2, b2, W3, b3,
           Wih0, Whh0, bih0, bhh0, Wih1, Whh1, bih1, bhh1,
           Wih2, Whh2, bih2, bhh2, Wlin, blin):
    ei = edge_index.astype(jnp.int32)
    pad = jnp.full((EP - E,), N, jnp.int32)
    src = jnp.concatenate([ei[0], pad]).reshape(NW, NCH, CHUNK)
    dst = jnp.concatenate([ei[1], pad]).reshape(NW, NCH, CHUNK)

    x_pad = jnp.concatenate([x, jnp.zeros((NP - N, D), x.dtype)])
    ones_c = jnp.ones((CHUNK,), jnp.float32)
    zeros_r = jnp.zeros((RPT,), jnp.float32)
    zeros_rh = jnp.zeros((RPT, PW), jnp.float32)

    deg = _sc_degree(dst, ones_c, zeros_r)

    t_shape = jax.ShapeDtypeStruct((NPASS, NP, PW), jnp.float32)
    sb = _spec_full((1, D))
    sw = _spec_full((D, D))
    dinv, t1 = _tc_call(
        _tc_pre_kernel,
        [_spec_deg, _spec_rows, sw],
        (_spec_rows1, _spec_t),
        (jax.ShapeDtypeStruct((NP, 1), jnp.float32), t_shape),
        deg, x_pad, W1)

    mid_in = [_spec_p, _spec_t, _spec_rows1, sb, sw]
    p1 = _sc_spmm(t1, src, dst, zeros_rh)
    t2 = _tc_call(_tc_mid_kernel, mid_in, _spec_t, t_shape,
                  p1, t1, dinv, b1.reshape(1, D), W2)

    p2 = _sc_spmm(t2, src, dst, zeros_rh)
    t3 = _tc_call(_tc_mid_kernel, mid_in, _spec_t, t_shape,
                  p2, t2, dinv, b2.reshape(1, D), W3)

    p3 = _sc_spmm(t3, src, dst, zeros_rh)
    swih = _spec_full((4 * D, D))
    sbl = _spec_full((1, 4 * D))
    out = _tc_call(
        _tc_post_kernel,
        [_spec_p, _spec_t, _spec_rows1, sb,
         swih, sbl, swih, sbl, swih, sbl, sw, sb],
        _spec_rows,
        jax.ShapeDtypeStruct((NP, D), jnp.float32),
        p3, t3, dinv, b3.reshape(1, D),
        Wih0, (bih0 + bhh0).reshape(1, 4 * D),
        Wih1, (bih1 + bhh1).reshape(1, 4 * D),
        Wih2, (bih2 + bhh2).reshape(1, 4 * D),
        Wlin, blin.reshape(1, D))

    return out[:N]


# speedup vs baseline: 15.8773x; 1.0454x over previous
import functools

import jax
import jax.numpy as jnp
from jax import lax
from jax.experimental import pallas as pl
from jax.experimental.pallas import tpu as pltpu
from jax.experimental.pallas import tpu_sc as plsc

N = 10000
D = 128
NPASS = 4
PW = D // NPASS
E = 320000
NTILES = 16
NCORES = 2
NW = NTILES * NCORES
CHUNK = 128
NCH = 80
NG = 8
NGRP = NCH // NG
EP = NW * NCH * CHUNK
NP = 10240
RPT = NP // NTILES

_mesh = plsc.VectorSubcoreMesh(core_axis_name="c", subcore_axis_name="s")


@functools.partial(
    pl.kernel,
    out_type=jax.ShapeDtypeStruct((NCORES, NP), jnp.float32),
    mesh=_mesh,
    scratch_types=[
        pltpu.VMEM((NCH, CHUNK), jnp.int32),
        pltpu.VMEM((CHUNK,), jnp.float32),
        pltpu.VMEM((RPT,), jnp.float32),
        pltpu.VMEM_SHARED((NP,), jnp.float32),
    ],
)
def _sc_degree(dst_hbm, ones_hbm, zeros_hbm, deg_out, dst_v, ones_v, zer_v, acc):
    cid = lax.axis_index("c")
    sid = lax.axis_index("s")
    wid = sid * NCORES + cid
    pltpu.sync_copy(dst_hbm.at[wid], dst_v)
    pltpu.sync_copy(ones_hbm, ones_v)
    pltpu.sync_copy(zeros_hbm, zer_v)
    pltpu.sync_copy(zer_v, acc.at[pl.ds(sid * RPT, RPT)])
    plsc.subcore_barrier()

    @pl.loop(0, NCH)
    def _(j):
        pltpu.sync_copy(ones_v, acc.at[dst_v.at[j]], add=True)

    plsc.subcore_barrier()
    pltpu.sync_copy(acc.at[pl.ds(sid * RPT, RPT)],
                    deg_out.at[cid, pl.ds(sid * RPT, RPT)])


@functools.partial(
    pl.kernel,
    out_type=jax.ShapeDtypeStruct((NCORES, NPASS, NP, PW), jnp.float32),
    mesh=_mesh,
    compiler_params=pltpu.CompilerParams(use_tc_tiling_on_sc=False),
    scratch_types=[
        pltpu.VMEM((NCH, CHUNK), jnp.int32),
        pltpu.VMEM((NCH, CHUNK), jnp.int32),
        pltpu.VMEM((NG * CHUNK, PW), jnp.float32),
        pltpu.VMEM((NG * CHUNK, PW), jnp.float32),
        pltpu.VMEM_SHARED((NP, PW), jnp.float32),
        pltpu.VMEM_SHARED((NP, PW), jnp.float32),
        pltpu.SemaphoreType.DMA,
        pltpu.SemaphoreType.DMA,
        pltpu.SemaphoreType.DMA,
        pltpu.SemaphoreType.DMA,
    ],
)
def _sc_spmm(t_hbm, src_hbm, dst_hbm, zeros_hbm, out,
             src_v, dst_v, buf0, buf1, acc, tsp, gs0, gs1, ss0, ss1):
    cid = lax.axis_index("c")
    sid = lax.axis_index("s")
    wid = sid * NCORES + cid
    pltpu.sync_copy(src_hbm.at[wid], src_v)
    pltpu.sync_copy(dst_hbm.at[wid], dst_v)

    bufs = (buf0, buf1)
    gsems = (gs0, gs1)
    ssems = (ss0, ss1)
    slab = pl.ds(sid * RPT, RPT)

    def fire_gathers(g, b):
        for i in range(NG):
            pltpu.async_copy(tsp.at[src_v.at[g * NG + i]],
                             bufs[b].at[pl.ds(i * CHUNK, CHUNK)], gsems[b])

    def drain(sem, buf):
        for i in range(NG):
            pltpu.make_async_copy(tsp.at[src_v.at[0]],
                                  buf.at[pl.ds(i * CHUNK, CHUNK)], sem).wait()

    def fire_scatters(g, b):
        for i in range(NG):
            pltpu.async_copy(bufs[b].at[pl.ds(i * CHUNK, CHUNK)],
                             acc.at[dst_v.at[g * NG + i]], ssems[b], add=True)

    for h in range(NPASS):
        pltpu.sync_copy(t_hbm.at[h, slab], tsp.at[slab])
        pltpu.sync_copy(zeros_hbm, acc.at[slab])
        plsc.subcore_barrier()
        fire_gathers(0, 0)

        @pl.loop(0, NGRP, step=2)
        def _(g):
            for b in range(2):
                gb = g + b
                @pl.when(gb >= 1)
                def _():
                    drain(ssems[1 - b], bufs[1 - b])

                @pl.when(gb + 1 < NGRP)
                def _():
                    fire_gathers(gb + 1, 1 - b)

                drain(gsems[b], bufs[b])
                fire_scatters(gb, b)

        drain(ssems[(NGRP - 1) % 2], bufs[(NGRP - 1) % 2])
        plsc.subcore_barrier()
        pltpu.sync_copy(acc.at[slab], out.at[cid, h, slab])


def _dot_t(a, w):
    return lax.dot_general(a, w, (((1,), (1,)), ((), ())),
                           preferred_element_type=jnp.float32)


def _split_store(t_ref, t_full):
    for h in range(NPASS):
        t_ref[h] = t_full[:, h * PW:(h + 1) * PW]


def _merge(p_ref, t_ref):
    p = jnp.concatenate([p_ref[0, h] + p_ref[1, h] for h in range(NPASS)],
                        axis=1)
    t = jnp.concatenate([t_ref[h] for h in range(NPASS)], axis=1)
    return p + t


def _tc_pre_kernel(deg_ref, x_ref, w1_ref, dinv_ref, t1_ref):
    deg = deg_ref[0] + deg_ref[1] + 1.0
    row = pl.program_id(0) * BR + lax.broadcasted_iota(jnp.int32, (BR, 1), 0)
    dinv = jnp.where(row < N, lax.rsqrt(deg)[:, None], 0.0)
    dinv_ref[...] = dinv
    _split_store(t1_ref, _dot_t(x_ref[...], w1_ref[...]) * dinv)


def _tc_mid_kernel(p_ref, t_ref, dinv_ref, b_ref, w_ref, tn_ref):
    dinv = dinv_ref[...]
    z = jax.nn.relu(dinv * _merge(p_ref, t_ref) + b_ref[...])
    _split_store(tn_ref, _dot_t(z, w_ref[...]) * dinv)


def _tc_post_kernel(p_ref, t_ref, dinv_ref, b3_ref,
                    wih0_ref, bl0_ref, wih1_ref, bl1_ref, wih2_ref, bl2_ref,
                    wlin_ref, blin_ref, out_ref):
    dinv = dinv_ref[...]
    h = jax.nn.relu(dinv * _merge(p_ref, t_ref) + b3_ref[...])
    for wih_ref, bl_ref in ((wih0_ref, bl0_ref), (wih1_ref, bl1_ref),
                            (wih2_ref, bl2_ref)):
        gates = _dot_t(h, wih_ref[...]) + bl_ref[...]
        i = jax.nn.sigmoid(gates[:, 0 * D:1 * D])
        g = jnp.tanh(gates[:, 2 * D:3 * D])
        o = jax.nn.sigmoid(gates[:, 3 * D:4 * D])
        h = o * jnp.tanh(i * g)
    out_ref[...] = _dot_t(h, wlin_ref[...]) + blin_ref[...]


BR = 2048

_spec_rows = pl.BlockSpec((BR, D), lambda i: (i, 0))
_spec_rows1 = pl.BlockSpec((BR, 1), lambda i: (i, 0))
_spec_deg = pl.BlockSpec((NCORES, BR), lambda i: (0, i))
_spec_t = pl.BlockSpec((NPASS, BR, PW), lambda i: (0, i, 0))
_spec_p = pl.BlockSpec((NCORES, NPASS, BR, PW), lambda i: (0, 0, i, 0))


def _spec_full(shape):
    return pl.BlockSpec(shape, lambda i: tuple(0 for _ in shape))


def _tc_call(body, in_specs, out_specs, out_shapes, *args):
    return pl.pallas_call(
        body,
        grid=(NP // BR,),
        in_specs=in_specs,
        out_specs=out_specs,
        out_shape=out_shapes,
    )(*args)


def kernel(x, edge_index, W1, b1, W2, b2, W3, b3,
           Wih0, Whh0, bih0, bhh0, Wih1, Whh1, bih1, bhh1,
           Wih2, Whh2, bih2, bhh2, Wlin, blin):
    ei = edge_index.astype(jnp.int32)
    pad = jnp.full((EP - E,), N, jnp.int32)
    src = jnp.concatenate([ei[0], pad]).reshape(NW, NCH, CHUNK)
    dst = jnp.concatenate([ei[1], pad]).reshape(NW, NCH, CHUNK)

    x_pad = jnp.concatenate([x, jnp.zeros((NP - N, D), x.dtype)])
    ones_c = jnp.ones((CHUNK,), jnp.float32)
    zeros_r = jnp.zeros((RPT,), jnp.float32)
    zeros_rh = jnp.zeros((RPT, PW), jnp.float32)

    deg = _sc_degree(dst, ones_c, zeros_r)

    t_shape = jax.ShapeDtypeStruct((NPASS, NP, PW), jnp.float32)
    sb = _spec_full((1, D))
    sw = _spec_full((D, D))
    dinv, t1 = _tc_call(
        _tc_pre_kernel,
        [_spec_deg, _spec_rows, sw],
        (_spec_rows1, _spec_t),
        (jax.ShapeDtypeStruct((NP, 1), jnp.float32), t_shape),
        deg, x_pad, W1)

    mid_in = [_spec_p, _spec_t, _spec_rows1, sb, sw]
    p1 = _sc_spmm(t1, src, dst, zeros_rh)
    t2 = _tc_call(_tc_mid_kernel, mid_in, _spec_t, t_shape,
                  p1, t1, dinv, b1.reshape(1, D), W2)

    p2 = _sc_spmm(t2, src, dst, zeros_rh)
    t3 = _tc_call(_tc_mid_kernel, mid_in, _spec_t, t_shape,
                  p2, t2, dinv, b2.reshape(1, D), W3)

    p3 = _sc_spmm(t3, src, dst, zeros_rh)
    swih = _spec_full((4 * D, D))
    sbl = _spec_full((1, 4 * D))
    out = _tc_call(
        _tc_post_kernel,
        [_spec_p, _spec_t, _spec_rows1, sb,
         swih, sbl, swih, sbl, swih, sbl, sw, sb],
        _spec_rows,
        jax.ShapeDtypeStruct((NP, D), jnp.float32),
        p3, t3, dinv, b3.reshape(1, D),
        Wih0, (bih0 + bhh0).reshape(1, 4 * D),
        Wih1, (bih1 + bhh1).reshape(1, 4 * D),
        Wih2, (bih2 + bhh2).reshape(1, 4 * D),
        Wlin, blin.reshape(1, D))

    return out[:N]


# speedup vs baseline: 17.8604x; 1.1249x over previous
import functools

import jax
import jax.numpy as jnp
from jax import lax
from jax.experimental import pallas as pl
from jax.experimental.pallas import tpu as pltpu
from jax.experimental.pallas import tpu_sc as plsc

N = 10000
D = 128
NPASS = 4
PW = D // NPASS
E = 320000
NTILES = 16
NCORES = 2
NW = NTILES * NCORES
CHUNK = 128
NCH = 80
NG = 4
NCHT = 160
NGRPT = NCHT // NG
EP = NW * NCH * CHUNK
NP = 10240
RPT = NP // NTILES

_mesh = plsc.VectorSubcoreMesh(core_axis_name="c", subcore_axis_name="s")


@functools.partial(
    pl.kernel,
    out_type=jax.ShapeDtypeStruct((NCORES, NP), jnp.float32),
    mesh=_mesh,
    scratch_types=[
        pltpu.VMEM((NCH, CHUNK), jnp.int32),
        pltpu.VMEM((CHUNK,), jnp.float32),
        pltpu.VMEM((RPT,), jnp.float32),
        pltpu.VMEM_SHARED((NP,), jnp.float32),
    ],
)
def _sc_degree(dst_hbm, ones_hbm, zeros_hbm, deg_out, dst_v, ones_v, zer_v, acc):
    cid = lax.axis_index("c")
    sid = lax.axis_index("s")
    pltpu.sync_copy(dst_hbm.at[sid, pl.ds(cid * NCH, NCH)], dst_v)
    pltpu.sync_copy(ones_hbm, ones_v)
    pltpu.sync_copy(zeros_hbm, zer_v)
    pltpu.sync_copy(zer_v, acc.at[pl.ds(sid * RPT, RPT)])
    plsc.subcore_barrier()

    @pl.loop(0, NCH)
    def _(j):
        pltpu.sync_copy(ones_v, acc.at[dst_v.at[j]], add=True)

    plsc.subcore_barrier()
    pltpu.sync_copy(acc.at[pl.ds(sid * RPT, RPT)],
                    deg_out.at[cid, pl.ds(sid * RPT, RPT)])


@functools.partial(
    pl.kernel,
    out_type=jax.ShapeDtypeStruct((NPASS, NP, PW), jnp.float32),
    mesh=_mesh,
    compiler_params=pltpu.CompilerParams(use_tc_tiling_on_sc=False),
    scratch_types=[
        pltpu.VMEM((NCHT, CHUNK), jnp.int32),
        pltpu.VMEM((NCHT, CHUNK), jnp.int32),
        pltpu.VMEM((NG * CHUNK, PW), jnp.float32),
        pltpu.VMEM((NG * CHUNK, PW), jnp.float32),
        pltpu.VMEM_SHARED((NP, PW), jnp.float32),
        pltpu.VMEM_SHARED((NP, PW), jnp.float32),
        pltpu.SemaphoreType.DMA,
        pltpu.SemaphoreType.DMA,
        pltpu.SemaphoreType.DMA,
        pltpu.SemaphoreType.DMA,
    ],
)
def _sc_spmm(t_hbm, src_hbm, dst_hbm, zeros_hbm, out,
             src_v, dst_v, buf0, buf1, acc, tsp, gs0, gs1, ss0, ss1):
    cid = lax.axis_index("c")
    sid = lax.axis_index("s")
    pltpu.sync_copy(src_hbm.at[sid], src_v)
    pltpu.sync_copy(dst_hbm.at[sid], dst_v)

    bufs = (buf0, buf1)
    gsems = (gs0, gs1)
    ssems = (ss0, ss1)
    slab = pl.ds(sid * RPT, RPT)

    def fire_gathers(g, b):
        for i in range(NG):
            pltpu.async_copy(tsp.at[src_v.at[g * NG + i]],
                             bufs[b].at[pl.ds(i * CHUNK, CHUNK)], gsems[b])

    def drain(sem, buf):
        for i in range(NG):
            pltpu.make_async_copy(tsp.at[src_v.at[0]],
                                  buf.at[pl.ds(i * CHUNK, CHUNK)], sem).wait()

    def fire_scatters(g, b):
        for i in range(NG):
            pltpu.async_copy(bufs[b].at[pl.ds(i * CHUNK, CHUNK)],
                             acc.at[dst_v.at[g * NG + i]], ssems[b], add=True)

    for h in range(NPASS):
        @pl.when(h // (NPASS // NCORES) == cid)
        def _():
            pltpu.sync_copy(t_hbm.at[h, slab], tsp.at[slab])
            pltpu.sync_copy(zeros_hbm, acc.at[slab])
            plsc.subcore_barrier()
            fire_gathers(0, 0)

            @pl.loop(0, NGRPT, step=2)
            def _(g):
                for b in range(2):
                    gb = g + b
                    @pl.when(gb >= 1)
                    def _():
                        drain(ssems[1 - b], bufs[1 - b])

                    @pl.when(gb + 1 < NGRPT)
                    def _():
                        fire_gathers(gb + 1, 1 - b)

                    drain(gsems[b], bufs[b])
                    fire_scatters(gb, b)

            drain(ssems[(NGRPT - 1) % 2], bufs[(NGRPT - 1) % 2])
            plsc.subcore_barrier()
            pltpu.sync_copy(acc.at[slab], out.at[h, slab])


def _dot_t(a, w):
    return lax.dot_general(a, w, (((1,), (1,)), ((), ())),
                           preferred_element_type=jnp.float32)


def _split_store(t_ref, t_full):
    for h in range(NPASS):
        t_ref[h] = t_full[:, h * PW:(h + 1) * PW]


def _merge(p_ref, t_ref):
    p = jnp.concatenate([p_ref[h] for h in range(NPASS)], axis=1)
    t = jnp.concatenate([t_ref[h] for h in range(NPASS)], axis=1)
    return p + t


def _tc_pre_kernel(deg_ref, x_ref, w1_ref, dinv_ref, t1_ref):
    deg = deg_ref[0] + deg_ref[1] + 1.0
    row = pl.program_id(0) * BR + lax.broadcasted_iota(jnp.int32, (BR, 1), 0)
    dinv = jnp.where(row < N, lax.rsqrt(deg)[:, None], 0.0)
    dinv_ref[...] = dinv
    _split_store(t1_ref, _dot_t(x_ref[...], w1_ref[...]) * dinv)


def _tc_mid_kernel(p_ref, t_ref, dinv_ref, b_ref, w_ref, tn_ref):
    dinv = dinv_ref[...]
    z = jax.nn.relu(dinv * _merge(p_ref, t_ref) + b_ref[...])
    _split_store(tn_ref, _dot_t(z, w_ref[...]) * dinv)


def _tc_post_kernel(p_ref, t_ref, dinv_ref, b3_ref,
                    wih0_ref, bl0_ref, wih1_ref, bl1_ref, wih2_ref, bl2_ref,
                    wlin_ref, blin_ref, out_ref):
    dinv = dinv_ref[...]
    h = jax.nn.relu(dinv * _merge(p_ref, t_ref) + b3_ref[...])
    for wih_ref, bl_ref in ((wih0_ref, bl0_ref), (wih1_ref, bl1_ref),
                            (wih2_ref, bl2_ref)):
        gates = _dot_t(h, wih_ref[...]) + bl_ref[...]
        i = jax.nn.sigmoid(gates[:, 0 * D:1 * D])
        g = jnp.tanh(gates[:, 2 * D:3 * D])
        o = jax.nn.sigmoid(gates[:, 3 * D:4 * D])
        h = o * jnp.tanh(i * g)
    out_ref[...] = _dot_t(h, wlin_ref[...]) + blin_ref[...]


BR = 2048

_spec_rows = pl.BlockSpec((BR, D), lambda i: (i, 0))
_spec_rows1 = pl.BlockSpec((BR, 1), lambda i: (i, 0))
_spec_deg = pl.BlockSpec((NCORES, BR), lambda i: (0, i))
_spec_t = pl.BlockSpec((NPASS, BR, PW), lambda i: (0, i, 0))
_spec_p = _spec_t


def _spec_full(shape):
    return pl.BlockSpec(shape, lambda i: tuple(0 for _ in shape))


def _tc_call(body, in_specs, out_specs, out_shapes, *args):
    return pl.pallas_call(
        body,
        grid=(NP // BR,),
        in_specs=in_specs,
        out_specs=out_specs,
        out_shape=out_shapes,
    )(*args)


def kernel(x, edge_index, W1, b1, W2, b2, W3, b3,
           Wih0, Whh0, bih0, bhh0, Wih1, Whh1, bih1, bhh1,
           Wih2, Whh2, bih2, bhh2, Wlin, blin):
    ei = edge_index.astype(jnp.int32)
    pad = jnp.full((EP - E,), N, jnp.int32)
    src = jnp.concatenate([ei[0], pad]).reshape(NTILES, NCHT, CHUNK)
    dst = jnp.concatenate([ei[1], pad]).reshape(NTILES, NCHT, CHUNK)

    x_pad = jnp.concatenate([x, jnp.zeros((NP - N, D), x.dtype)])
    ones_c = jnp.ones((CHUNK,), jnp.float32)
    zeros_r = jnp.zeros((RPT,), jnp.float32)
    zeros_rh = jnp.zeros((RPT, PW), jnp.float32)

    deg = _sc_degree(dst, ones_c, zeros_r)

    t_shape = jax.ShapeDtypeStruct((NPASS, NP, PW), jnp.float32)
    sb = _spec_full((1, D))
    sw = _spec_full((D, D))
    dinv, t1 = _tc_call(
        _tc_pre_kernel,
        [_spec_deg, _spec_rows, sw],
        (_spec_rows1, _spec_t),
        (jax.ShapeDtypeStruct((NP, 1), jnp.float32), t_shape),
        deg, x_pad, W1)

    mid_in = [_spec_p, _spec_t, _spec_rows1, sb, sw]
    p1 = _sc_spmm(t1, src, dst, zeros_rh)
    t2 = _tc_call(_tc_mid_kernel, mid_in, _spec_t, t_shape,
                  p1, t1, dinv, b1.reshape(1, D), W2)

    p2 = _sc_spmm(t2, src, dst, zeros_rh)
    t3 = _tc_call(_tc_mid_kernel, mid_in, _spec_t, t_shape,
                  p2, t2, dinv, b2.reshape(1, D), W3)

    p3 = _sc_spmm(t3, src, dst, zeros_rh)
    swih = _spec_full((4 * D, D))
    sbl = _spec_full((1, 4 * D))
    out = _tc_call(
        _tc_post_kernel,
        [_spec_p, _spec_t, _spec_rows1, sb,
         swih, sbl, swih, sbl, swih, sbl, sw, sb],
        _spec_rows,
        jax.ShapeDtypeStruct((NP, D), jnp.float32),
        p3, t3, dinv, b3.reshape(1, D),
        Wih0, (bih0 + bhh0).reshape(1, 4 * D),
        Wih1, (bih1 + bhh1).reshape(1, 4 * D),
        Wih2, (bih2 + bhh2).reshape(1, 4 * D),
        Wlin, blin.reshape(1, D))

    return out[:N]


# speedup vs baseline: 17.9383x; 1.0044x over previous
import functools

import jax
import jax.numpy as jnp
from jax import lax
from jax.experimental import pallas as pl
from jax.experimental.pallas import tpu as pltpu
from jax.experimental.pallas import tpu_sc as plsc

N = 10000
D = 128
NPASS = 4
PW = D // NPASS
E = 320000
NTILES = 16
NCORES = 2
NW = NTILES * NCORES
CHUNK = 128
NCH = 80
NG = 4
NCHT = 160
NGRPT = NCHT // NG
EP = NW * NCH * CHUNK
NP = 10240
RPT = NP // NTILES

_mesh = plsc.VectorSubcoreMesh(core_axis_name="c", subcore_axis_name="s")


@functools.partial(
    pl.kernel,
    out_type=jax.ShapeDtypeStruct((NCORES, NP), jnp.float32),
    mesh=_mesh,
    scratch_types=[
        pltpu.VMEM((NCH, CHUNK), jnp.int32),
        pltpu.VMEM((CHUNK,), jnp.float32),
        pltpu.VMEM((RPT,), jnp.float32),
        pltpu.VMEM_SHARED((NP,), jnp.float32),
    ],
)
def _sc_degree(dst_hbm, ones_hbm, zeros_hbm, deg_out, dst_v, ones_v, zer_v, acc):
    cid = lax.axis_index("c")
    sid = lax.axis_index("s")
    pltpu.sync_copy(dst_hbm.at[sid, pl.ds(cid * NCH, NCH)], dst_v)
    pltpu.sync_copy(ones_hbm, ones_v)
    pltpu.sync_copy(zeros_hbm, zer_v)
    pltpu.sync_copy(zer_v, acc.at[pl.ds(sid * RPT, RPT)])
    plsc.subcore_barrier()

    @pl.loop(0, NCH)
    def _(j):
        pltpu.sync_copy(ones_v, acc.at[dst_v.at[j]], add=True)

    plsc.subcore_barrier()
    pltpu.sync_copy(acc.at[pl.ds(sid * RPT, RPT)],
                    deg_out.at[cid, pl.ds(sid * RPT, RPT)])


@functools.partial(
    pl.kernel,
    out_type=jax.ShapeDtypeStruct((NPASS, NP, PW), jnp.float32),
    mesh=_mesh,
    compiler_params=pltpu.CompilerParams(use_tc_tiling_on_sc=False),
    scratch_types=[
        pltpu.VMEM((NCHT, CHUNK), jnp.int32),
        pltpu.VMEM((NCHT, CHUNK), jnp.int32),
        pltpu.VMEM((NG * CHUNK, PW), jnp.float32),
        pltpu.VMEM((NG * CHUNK, PW), jnp.float32),
        pltpu.VMEM_SHARED((NP, PW), jnp.float32),
        pltpu.VMEM_SHARED((NP, PW), jnp.float32),
        pltpu.SemaphoreType.DMA,
        pltpu.SemaphoreType.DMA,
        pltpu.SemaphoreType.DMA,
        pltpu.SemaphoreType.DMA,
    ],
)
def _sc_spmm(t_hbm, src_hbm, dst_hbm, zeros_hbm, out,
             src_v, dst_v, buf0, buf1, acc, tsp, gs0, gs1, ss0, ss1):
    cid = lax.axis_index("c")
    sid = lax.axis_index("s")
    pltpu.sync_copy(src_hbm.at[sid], src_v)
    pltpu.sync_copy(dst_hbm.at[sid], dst_v)

    bufs = (buf0, buf1)
    gsems = (gs0, gs1)
    ssems = (ss0, ss1)
    slab = pl.ds(sid * RPT, RPT)

    def fire_gathers(g, b):
        for i in range(NG):
            pltpu.async_copy(tsp.at[src_v.at[g * NG + i]],
                             bufs[b].at[pl.ds(i * CHUNK, CHUNK)], gsems[b])

    def drain(sem, buf):
        for i in range(NG):
            pltpu.make_async_copy(tsp.at[src_v.at[0]],
                                  buf.at[pl.ds(i * CHUNK, CHUNK)], sem).wait()

    def fire_scatters(g, b):
        for i in range(NG):
            pltpu.async_copy(bufs[b].at[pl.ds(i * CHUNK, CHUNK)],
                             acc.at[dst_v.at[g * NG + i]], ssems[b], add=True)

    for h in range(NPASS):
        @pl.when(h // (NPASS // NCORES) == cid)
        def _():
            pltpu.sync_copy(t_hbm.at[h, slab], tsp.at[slab])
            pltpu.sync_copy(zeros_hbm, acc.at[slab])
            plsc.subcore_barrier()
            fire_gathers(0, 0)

            @pl.loop(0, NGRPT, step=2)
            def _(g):
                for b in range(2):
                    gb = g + b
                    @pl.when(gb >= 1)
                    def _():
                        drain(ssems[1 - b], bufs[1 - b])

                    @pl.when(gb + 1 < NGRPT)
                    def _():
                        fire_gathers(gb + 1, 1 - b)

                    drain(gsems[b], bufs[b])
                    fire_scatters(gb, b)

            drain(ssems[(NGRPT - 1) % 2], bufs[(NGRPT - 1) % 2])
            plsc.subcore_barrier()
            pltpu.sync_copy(acc.at[slab], out.at[h, slab])


def _dot_t(a, w):
    return lax.dot_general(a, w, (((1,), (1,)), ((), ())),
                           preferred_element_type=jnp.float32)


def _split_store(t_ref, t_full):
    for h in range(NPASS):
        t_ref[h] = t_full[:, h * PW:(h + 1) * PW]


def _merge(p_ref, t_ref):
    p = jnp.concatenate([p_ref[h] for h in range(NPASS)], axis=1)
    t = jnp.concatenate([t_ref[h] for h in range(NPASS)], axis=1)
    return p + t


def _tc_pre_a_kernel(x_ref, w1_ref, h_ref):
    h_ref[...] = _dot_t(x_ref[...], w1_ref[...])


def _tc_pre_b_kernel(deg_ref, h_ref, dinv_ref, t1_ref):
    deg = deg_ref[0] + deg_ref[1] + 1.0
    row = pl.program_id(0) * BR + lax.broadcasted_iota(jnp.int32, (BR, 1), 0)
    dinv = jnp.where(row < N, lax.rsqrt(deg)[:, None], 0.0)
    dinv_ref[...] = dinv
    _split_store(t1_ref, h_ref[...] * dinv)


def _tc_mid_kernel(p_ref, t_ref, dinv_ref, b_ref, w_ref, tn_ref):
    dinv = dinv_ref[...]
    z = jax.nn.relu(dinv * _merge(p_ref, t_ref) + b_ref[...])
    _split_store(tn_ref, _dot_t(z, w_ref[...]) * dinv)


def _tc_post_kernel(p_ref, t_ref, dinv_ref, b3_ref,
                    wih0_ref, bl0_ref, wih1_ref, bl1_ref, wih2_ref, bl2_ref,
                    wlin_ref, blin_ref, out_ref):
    dinv = dinv_ref[...]
    h = jax.nn.relu(dinv * _merge(p_ref, t_ref) + b3_ref[...])
    for wih_ref, bl_ref in ((wih0_ref, bl0_ref), (wih1_ref, bl1_ref),
                            (wih2_ref, bl2_ref)):
        gates = _dot_t(h, wih_ref[...]) + bl_ref[...]
        i = jax.nn.sigmoid(gates[:, 0 * D:1 * D])
        g = jnp.tanh(gates[:, 2 * D:3 * D])
        o = jax.nn.sigmoid(gates[:, 3 * D:4 * D])
        h = o * jnp.tanh(i * g)
    out_ref[...] = _dot_t(h, wlin_ref[...]) + blin_ref[...]


BR = 2048

_spec_rows = pl.BlockSpec((BR, D), lambda i: (i, 0))
_spec_rows1 = pl.BlockSpec((BR, 1), lambda i: (i, 0))
_spec_deg = pl.BlockSpec((NCORES, BR), lambda i: (0, i))
_spec_t = pl.BlockSpec((NPASS, BR, PW), lambda i: (0, i, 0))
_spec_p = _spec_t


def _spec_full(shape):
    return pl.BlockSpec(shape, lambda i: tuple(0 for _ in shape))


def _tc_call(body, in_specs, out_specs, out_shapes, *args):
    return pl.pallas_call(
        body,
        grid=(NP // BR,),
        in_specs=in_specs,
        out_specs=out_specs,
        out_shape=out_shapes,
    )(*args)


def kernel(x, edge_index, W1, b1, W2, b2, W3, b3,
           Wih0, Whh0, bih0, bhh0, Wih1, Whh1, bih1, bhh1,
           Wih2, Whh2, bih2, bhh2, Wlin, blin):
    ei = edge_index.astype(jnp.int32)
    pad = jnp.full((EP - E,), N, jnp.int32)
    src = jnp.concatenate([ei[0], pad]).reshape(NTILES, NCHT, CHUNK)
    dst = jnp.concatenate([ei[1], pad]).reshape(NTILES, NCHT, CHUNK)

    x_pad = jnp.concatenate([x, jnp.zeros((NP - N, D), x.dtype)])
    ones_c = jnp.ones((CHUNK,), jnp.float32)
    zeros_r = jnp.zeros((RPT,), jnp.float32)
    zeros_rh = jnp.zeros((RPT, PW), jnp.float32)

    deg = _sc_degree(dst, ones_c, zeros_r)

    t_shape = jax.ShapeDtypeStruct((NPASS, NP, PW), jnp.float32)
    sb = _spec_full((1, D))
    sw = _spec_full((D, D))
    h1 = _tc_call(
        _tc_pre_a_kernel, [_spec_rows, sw], _spec_rows,
        jax.ShapeDtypeStruct((NP, D), jnp.float32), x_pad, W1)
    dinv, t1 = _tc_call(
        _tc_pre_b_kernel,
        [_spec_deg, _spec_rows],
        (_spec_rows1, _spec_t),
        (jax.ShapeDtypeStruct((NP, 1), jnp.float32), t_shape),
        deg, h1)

    mid_in = [_spec_p, _spec_t, _spec_rows1, sb, sw]
    p1 = _sc_spmm(t1, src, dst, zeros_rh)
    t2 = _tc_call(_tc_mid_kernel, mid_in, _spec_t, t_shape,
                  p1, t1, dinv, b1.reshape(1, D), W2)

    p2 = _sc_spmm(t2, src, dst, zeros_rh)
    t3 = _tc_call(_tc_mid_kernel, mid_in, _spec_t, t_shape,
                  p2, t2, dinv, b2.reshape(1, D), W3)

    p3 = _sc_spmm(t3, src, dst, zeros_rh)
    swih = _spec_full((4 * D, D))
    sbl = _spec_full((1, 4 * D))
    out = _tc_call(
        _tc_post_kernel,
        [_spec_p, _spec_t, _spec_rows1, sb,
         swih, sbl, swih, sbl, swih, sbl, sw, sb],
        _spec_rows,
        jax.ShapeDtypeStruct((NP, D), jnp.float32),
        p3, t3, dinv, b3.reshape(1, D),
        Wih0, (bih0 + bhh0).reshape(1, 4 * D),
        Wih1, (bih1 + bhh1).reshape(1, 4 * D),
        Wih2, (bih2 + bhh2).reshape(1, 4 * D),
        Wlin, blin.reshape(1, D))

    return out[:N]


# speedup vs baseline: 18.0993x; 1.0090x over previous
import functools

import jax
import jax.numpy as jnp
from jax import lax
from jax.experimental import pallas as pl
from jax.experimental.pallas import tpu as pltpu
from jax.experimental.pallas import tpu_sc as plsc

N = 10000
D = 128
NPASS = 4
PW = D // NPASS
E = 320000
NTILES = 16
NCORES = 2
NW = NTILES * NCORES
CHUNK = 128
NCH = 80
NG = 5
NCHT = 160
NGRPT = NCHT // NG
EP = NW * NCH * CHUNK
NP = 10240
RPT = NP // NTILES

_mesh = plsc.VectorSubcoreMesh(core_axis_name="c", subcore_axis_name="s")


@functools.partial(
    pl.kernel,
    out_type=jax.ShapeDtypeStruct((NCORES, NP), jnp.float32),
    mesh=_mesh,
    scratch_types=[
        pltpu.VMEM((NCH, CHUNK), jnp.int32),
        pltpu.VMEM((CHUNK,), jnp.float32),
        pltpu.VMEM((RPT,), jnp.float32),
        pltpu.VMEM_SHARED((NP,), jnp.float32),
    ],
)
def _sc_degree(dst_hbm, ones_hbm, zeros_hbm, deg_out, dst_v, ones_v, zer_v, acc):
    cid = lax.axis_index("c")
    sid = lax.axis_index("s")
    pltpu.sync_copy(dst_hbm.at[sid, pl.ds(cid * NCH, NCH)], dst_v)
    pltpu.sync_copy(ones_hbm, ones_v)
    pltpu.sync_copy(zeros_hbm, zer_v)
    pltpu.sync_copy(zer_v, acc.at[pl.ds(sid * RPT, RPT)])
    plsc.subcore_barrier()

    @pl.loop(0, NCH)
    def _(j):
        pltpu.sync_copy(ones_v, acc.at[dst_v.at[j]], add=True)

    plsc.subcore_barrier()
    pltpu.sync_copy(acc.at[pl.ds(sid * RPT, RPT)],
                    deg_out.at[cid, pl.ds(sid * RPT, RPT)])


@functools.partial(
    pl.kernel,
    out_type=jax.ShapeDtypeStruct((NPASS, NP, PW), jnp.float32),
    mesh=_mesh,
    compiler_params=pltpu.CompilerParams(use_tc_tiling_on_sc=False),
    scratch_types=[
        pltpu.VMEM((NCHT, CHUNK), jnp.int32),
        pltpu.VMEM((NCHT, CHUNK), jnp.int32),
        pltpu.VMEM((NG * CHUNK, PW), jnp.float32),
        pltpu.VMEM((NG * CHUNK, PW), jnp.float32),
        pltpu.VMEM_SHARED((NP, PW), jnp.float32),
        pltpu.VMEM_SHARED((NP, PW), jnp.float32),
        pltpu.SemaphoreType.DMA,
        pltpu.SemaphoreType.DMA,
        pltpu.SemaphoreType.DMA,
        pltpu.SemaphoreType.DMA,
    ],
)
def _sc_spmm(t_hbm, src_hbm, dst_hbm, zeros_hbm, out,
             src_v, dst_v, buf0, buf1, acc, tsp, gs0, gs1, ss0, ss1):
    cid = lax.axis_index("c")
    sid = lax.axis_index("s")
    pltpu.sync_copy(src_hbm.at[sid], src_v)
    pltpu.sync_copy(dst_hbm.at[sid], dst_v)

    bufs = (buf0, buf1)
    gsems = (gs0, gs1)
    ssems = (ss0, ss1)
    slab = pl.ds(sid * RPT, RPT)

    def fire_gathers(g, b):
        for i in range(NG):
            pltpu.async_copy(tsp.at[src_v.at[g * NG + i]],
                             bufs[b].at[pl.ds(i * CHUNK, CHUNK)], gsems[b])

    def drain(sem, buf):
        for i in range(NG):
            pltpu.make_async_copy(tsp.at[src_v.at[0]],
                                  buf.at[pl.ds(i * CHUNK, CHUNK)], sem).wait()

    def fire_scatters(g, b):
        for i in range(NG):
            pltpu.async_copy(bufs[b].at[pl.ds(i * CHUNK, CHUNK)],
                             acc.at[dst_v.at[g * NG + i]], ssems[b], add=True)

    for h in range(NPASS):
        @pl.when(h // (NPASS // NCORES) == cid)
        def _():
            pltpu.sync_copy(t_hbm.at[h, slab], tsp.at[slab])
            pltpu.sync_copy(zeros_hbm, acc.at[slab])
            plsc.subcore_barrier()
            fire_gathers(0, 0)

            @pl.loop(0, NGRPT, step=2)
            def _(g):
                for b in range(2):
                    gb = g + b
                    @pl.when(gb >= 1)
                    def _():
                        drain(ssems[1 - b], bufs[1 - b])

                    @pl.when(gb + 1 < NGRPT)
                    def _():
                        fire_gathers(gb + 1, 1 - b)

                    drain(gsems[b], bufs[b])
                    fire_scatters(gb, b)

            drain(ssems[(NGRPT - 1) % 2], bufs[(NGRPT - 1) % 2])
            plsc.subcore_barrier()
            pltpu.sync_copy(acc.at[slab], out.at[h, slab])


def _dot_t(a, w):
    return lax.dot_general(a, w, (((1,), (1,)), ((), ())),
                           preferred_element_type=jnp.float32)


def _split_store(t_ref, t_full):
    for h in range(NPASS):
        t_ref[h] = t_full[:, h * PW:(h + 1) * PW]


def _merge(p_ref, t_ref):
    p = jnp.concatenate([p_ref[h] for h in range(NPASS)], axis=1)
    t = jnp.concatenate([t_ref[h] for h in range(NPASS)], axis=1)
    return p + t


def _tc_pre_a_kernel(x_ref, w1_ref, h_ref):
    h_ref[...] = _dot_t(x_ref[...], w1_ref[...])


def _tc_pre_b_kernel(deg_ref, h_ref, dinv_ref, t1_ref):
    deg = deg_ref[0] + deg_ref[1] + 1.0
    row = pl.program_id(0) * BR + lax.broadcasted_iota(jnp.int32, (BR, 1), 0)
    dinv = jnp.where(row < N, lax.rsqrt(deg)[:, None], 0.0)
    dinv_ref[...] = dinv
    _split_store(t1_ref, h_ref[...] * dinv)


def _tc_mid_kernel(p_ref, t_ref, dinv_ref, b_ref, w_ref, tn_ref):
    dinv = dinv_ref[...]
    z = jax.nn.relu(dinv * _merge(p_ref, t_ref) + b_ref[...])
    _split_store(tn_ref, _dot_t(z, w_ref[...]) * dinv)


def _tc_post_kernel(p_ref, t_ref, dinv_ref, b3_ref,
                    wih0_ref, bl0_ref, wih1_ref, bl1_ref, wih2_ref, bl2_ref,
                    wlin_ref, blin_ref, out_ref):
    dinv = dinv_ref[...]
    h = jax.nn.relu(dinv * _merge(p_ref, t_ref) + b3_ref[...])
    for wih_ref, bl_ref in ((wih0_ref, bl0_ref), (wih1_ref, bl1_ref),
                            (wih2_ref, bl2_ref)):
        gates = _dot_t(h, wih_ref[...]) + bl_ref[...]
        i = jax.nn.sigmoid(gates[:, 0 * D:1 * D])
        g = jnp.tanh(gates[:, 2 * D:3 * D])
        o = jax.nn.sigmoid(gates[:, 3 * D:4 * D])
        h = o * jnp.tanh(i * g)
    out_ref[...] = _dot_t(h, wlin_ref[...]) + blin_ref[...]


BR = 2048

_spec_rows = pl.BlockSpec((BR, D), lambda i: (i, 0))
_spec_rows1 = pl.BlockSpec((BR, 1), lambda i: (i, 0))
_spec_deg = pl.BlockSpec((NCORES, BR), lambda i: (0, i))
_spec_t = pl.BlockSpec((NPASS, BR, PW), lambda i: (0, i, 0))
_spec_p = _spec_t


def _spec_full(shape):
    return pl.BlockSpec(shape, lambda i: tuple(0 for _ in shape))


def _tc_call(body, in_specs, out_specs, out_shapes, *args):
    return pl.pallas_call(
        body,
        grid=(NP // BR,),
        in_specs=in_specs,
        out_specs=out_specs,
        out_shape=out_shapes,
    )(*args)


def kernel(x, edge_index, W1, b1, W2, b2, W3, b3,
           Wih0, Whh0, bih0, bhh0, Wih1, Whh1, bih1, bhh1,
           Wih2, Whh2, bih2, bhh2, Wlin, blin):
    ei = edge_index.astype(jnp.int32)
    pad = jnp.full((EP - E,), N, jnp.int32)
    src = jnp.concatenate([ei[0], pad]).reshape(NTILES, NCHT, CHUNK)
    dst = jnp.concatenate([ei[1], pad]).reshape(NTILES, NCHT, CHUNK)

    x_pad = jnp.concatenate([x, jnp.zeros((NP - N, D), x.dtype)])
    ones_c = jnp.ones((CHUNK,), jnp.float32)
    zeros_r = jnp.zeros((RPT,), jnp.float32)
    zeros_rh = jnp.zeros((RPT, PW), jnp.float32)

    deg = _sc_degree(dst, ones_c, zeros_r)

    t_shape = jax.ShapeDtypeStruct((NPASS, NP, PW), jnp.float32)
    sb = _spec_full((1, D))
    sw = _spec_full((D, D))
    h1 = _tc_call(
        _tc_pre_a_kernel, [_spec_rows, sw], _spec_rows,
        jax.ShapeDtypeStruct((NP, D), jnp.float32), x_pad, W1)
    dinv, t1 = _tc_call(
        _tc_pre_b_kernel,
        [_spec_deg, _spec_rows],
        (_spec_rows1, _spec_t),
        (jax.ShapeDtypeStruct((NP, 1), jnp.float32), t_shape),
        deg, h1)

    mid_in = [_spec_p, _spec_t, _spec_rows1, sb, sw]
    p1 = _sc_spmm(t1, src, dst, zeros_rh)
    t2 = _tc_call(_tc_mid_kernel, mid_in, _spec_t, t_shape,
                  p1, t1, dinv, b1.reshape(1, D), W2)

    p2 = _sc_spmm(t2, src, dst, zeros_rh)
    t3 = _tc_call(_tc_mid_kernel, mid_in, _spec_t, t_shape,
                  p2, t2, dinv, b2.reshape(1, D), W3)

    p3 = _sc_spmm(t3, src, dst, zeros_rh)
    swih = _spec_full((4 * D, D))
    sbl = _spec_full((1, 4 * D))
    out = _tc_call(
        _tc_post_kernel,
        [_spec_p, _spec_t, _spec_rows1, sb,
         swih, sbl, swih, sbl, swih, sbl, sw, sb],
        _spec_rows,
        jax.ShapeDtypeStruct((NP, D), jnp.float32),
        p3, t3, dinv, b3.reshape(1, D),
        Wih0, (bih0 + bhh0).reshape(1, 4 * D),
        Wih1, (bih1 + bhh1).reshape(1, 4 * D),
        Wih2, (bih2 + bhh2).reshape(1, 4 * D),
        Wlin, blin.reshape(1, D))

    return out[:N]


# speedup vs baseline: 18.1981x; 1.0055x over previous
import functools

import jax
import jax.numpy as jnp
from jax import lax
from jax.experimental import pallas as pl
from jax.experimental.pallas import tpu as pltpu
from jax.experimental.pallas import tpu_sc as plsc

N = 10000
D = 128
NPASS = 4
PW = D // NPASS
E = 320000
NTILES = 16
NCORES = 2
NW = NTILES * NCORES
CHUNK = 128
NCH = 80
NG = 5
NCHT = 160
NGRPT = NCHT // NG
EP = NW * NCH * CHUNK
NP = 10240
RPT = NP // NTILES

_mesh = plsc.VectorSubcoreMesh(core_axis_name="c", subcore_axis_name="s")


@functools.partial(
    pl.kernel,
    out_type=jax.ShapeDtypeStruct((NCORES, NP), jnp.float32),
    mesh=_mesh,
    scratch_types=[
        pltpu.VMEM((NCH, CHUNK), jnp.int32),
        pltpu.VMEM((CHUNK,), jnp.float32),
        pltpu.VMEM((RPT,), jnp.float32),
        pltpu.VMEM_SHARED((NP,), jnp.float32),
    ],
)
def _sc_degree(dst_hbm, ones_hbm, zeros_hbm, deg_out, dst_v, ones_v, zer_v, acc):
    cid = lax.axis_index("c")
    sid = lax.axis_index("s")
    pltpu.sync_copy(dst_hbm.at[sid, pl.ds(cid * NCH, NCH)], dst_v)
    pltpu.sync_copy(ones_hbm, ones_v)
    pltpu.sync_copy(zeros_hbm, zer_v)
    pltpu.sync_copy(zer_v, acc.at[pl.ds(sid * RPT, RPT)])
    plsc.subcore_barrier()

    @pl.loop(0, NCH)
    def _(j):
        pltpu.sync_copy(ones_v, acc.at[dst_v.at[j]], add=True)

    plsc.subcore_barrier()
    pltpu.sync_copy(acc.at[pl.ds(sid * RPT, RPT)],
                    deg_out.at[cid, pl.ds(sid * RPT, RPT)])


@functools.partial(
    pl.kernel,
    out_type=jax.ShapeDtypeStruct((NPASS, NP, PW), jnp.float32),
    mesh=_mesh,
    compiler_params=pltpu.CompilerParams(use_tc_tiling_on_sc=False),
    scratch_types=[
        pltpu.VMEM((NCHT, CHUNK), jnp.int32),
        pltpu.VMEM((NCHT, CHUNK), jnp.int32),
        pltpu.VMEM((NG * CHUNK, PW), jnp.float32),
        pltpu.VMEM((NG * CHUNK, PW), jnp.float32),
        pltpu.VMEM_SHARED((NP, PW), jnp.float32),
        pltpu.VMEM_SHARED((NP, PW), jnp.float32),
        pltpu.SemaphoreType.DMA,
        pltpu.SemaphoreType.DMA,
        pltpu.SemaphoreType.DMA,
        pltpu.SemaphoreType.DMA,
    ],
)
def _sc_spmm(t_hbm, src_hbm, dst_hbm, zeros_hbm, out,
             src_v, dst_v, buf0, buf1, acc, tsp, gs0, gs1, ss0, ss1):
    cid = lax.axis_index("c")
    sid = lax.axis_index("s")
    pltpu.sync_copy(src_hbm.at[sid], src_v)
    pltpu.sync_copy(dst_hbm.at[sid], dst_v)

    bufs = (buf0, buf1)
    gsems = (gs0, gs1)
    ssems = (ss0, ss1)
    slab = pl.ds(sid * RPT, RPT)

    def fire_gathers(g, b):
        for i in range(NG):
            pltpu.async_copy(tsp.at[src_v.at[g * NG + i]],
                             bufs[b].at[pl.ds(i * CHUNK, CHUNK)], gsems[b])

    def drain(sem, buf):
        for i in range(NG):
            pltpu.make_async_copy(tsp.at[src_v.at[0]],
                                  buf.at[pl.ds(i * CHUNK, CHUNK)], sem).wait()

    def fire_scatters(g, b):
        for i in range(NG):
            pltpu.async_copy(bufs[b].at[pl.ds(i * CHUNK, CHUNK)],
                             acc.at[dst_v.at[g * NG + i]], ssems[b], add=True)

    for h in range(NPASS):
        @pl.when(h // (NPASS // NCORES) == cid)
        def _():
            pltpu.sync_copy(t_hbm.at[h, slab], tsp.at[slab])
            pltpu.sync_copy(zeros_hbm, acc.at[slab])
            plsc.subcore_barrier()
            fire_gathers(0, 0)

            @pl.loop(0, NGRPT, step=2)
            def _(g):
                for b in range(2):
                    gb = g + b
                    @pl.when(gb >= 1)
                    def _():
                        drain(ssems[1 - b], bufs[1 - b])

                    @pl.when(gb + 1 < NGRPT)
                    def _():
                        fire_gathers(gb + 1, 1 - b)

                    drain(gsems[b], bufs[b])
                    fire_scatters(gb, b)

            drain(ssems[(NGRPT - 1) % 2], bufs[(NGRPT - 1) % 2])
            plsc.subcore_barrier()
            pltpu.sync_copy(acc.at[slab], out.at[h, slab])


def _dot_t(a, w):
    return lax.dot_general(a, w, (((1,), (1,)), ((), ())),
                           preferred_element_type=jnp.float32)


def _split_store(t_ref, t_full):
    for h in range(NPASS):
        t_ref[h] = t_full[:, h * PW:(h + 1) * PW]


def _merge(p_ref, t_ref):
    p = jnp.concatenate([p_ref[h] for h in range(NPASS)], axis=1)
    t = jnp.concatenate([t_ref[h] for h in range(NPASS)], axis=1)
    return p + t


def _tc_pre_a_kernel(x_ref, w1_ref, h_ref):
    h_ref[...] = _dot_t(x_ref[...], w1_ref[...])


def _tc_pre_b_kernel(deg_ref, h_ref, dinv_ref, t1_ref):
    deg = deg_ref[0] + deg_ref[1] + 1.0
    row = pl.program_id(0) * BR + lax.broadcasted_iota(jnp.int32, (BR, 1), 0)
    dinv = jnp.where(row < N, lax.rsqrt(deg)[:, None], 0.0)
    dinv_ref[...] = dinv
    _split_store(t1_ref, h_ref[...] * dinv)


def _tc_mid_kernel(p_ref, t_ref, dinv_ref, b_ref, w_ref, tn_ref):
    dinv = dinv_ref[...]
    z = jax.nn.relu(dinv * _merge(p_ref, t_ref) + b_ref[...])
    _split_store(tn_ref, _dot_t(z, w_ref[...]) * dinv)


def _tc_post_kernel(p_ref, t_ref, dinv_ref, b3_ref,
                    wih0_ref, bl0_ref, wih1_ref, bl1_ref, wih2_ref, bl2_ref,
                    wlin_ref, blin_ref, out_ref):
    dinv = dinv_ref[...]
    h = jax.nn.relu(dinv * _merge(p_ref, t_ref) + b3_ref[...])
    for wih_ref, bl_ref in ((wih0_ref, bl0_ref), (wih1_ref, bl1_ref),
                            (wih2_ref, bl2_ref)):
        gates = _dot_t(h, wih_ref[...]) + bl_ref[...]
        i = jax.nn.sigmoid(gates[:, 0 * D:1 * D])
        g = jnp.tanh(gates[:, 2 * D:3 * D])
        o = jax.nn.sigmoid(gates[:, 3 * D:4 * D])
        h = o * jnp.tanh(i * g)
    out_ref[...] = _dot_t(h, wlin_ref[...]) + blin_ref[...]


BR = 2560

_spec_rows = pl.BlockSpec((BR, D), lambda i: (i, 0))
_spec_rows1 = pl.BlockSpec((BR, 1), lambda i: (i, 0))
_spec_deg = pl.BlockSpec((NCORES, BR), lambda i: (0, i))
_spec_t = pl.BlockSpec((NPASS, BR, PW), lambda i: (0, i, 0))
_spec_p = _spec_t


def _spec_full(shape):
    return pl.BlockSpec(shape, lambda i: tuple(0 for _ in shape))


def _tc_call(body, in_specs, out_specs, out_shapes, *args):
    return pl.pallas_call(
        body,
        grid=(NP // BR,),
        in_specs=in_specs,
        out_specs=out_specs,
        out_shape=out_shapes,
    )(*args)


def kernel(x, edge_index, W1, b1, W2, b2, W3, b3,
           Wih0, Whh0, bih0, bhh0, Wih1, Whh1, bih1, bhh1,
           Wih2, Whh2, bih2, bhh2, Wlin, blin):
    ei = edge_index.astype(jnp.int32)
    pad = jnp.full((EP - E,), N, jnp.int32)
    src = jnp.concatenate([ei[0], pad]).reshape(NTILES, NCHT, CHUNK)
    dst = jnp.concatenate([ei[1], pad]).reshape(NTILES, NCHT, CHUNK)

    x_pad = jnp.concatenate([x, jnp.zeros((NP - N, D), x.dtype)])
    ones_c = jnp.ones((CHUNK,), jnp.float32)
    zeros_r = jnp.zeros((RPT,), jnp.float32)
    zeros_rh = jnp.zeros((RPT, PW), jnp.float32)

    deg = _sc_degree(dst, ones_c, zeros_r)

    t_shape = jax.ShapeDtypeStruct((NPASS, NP, PW), jnp.float32)
    sb = _spec_full((1, D))
    sw = _spec_full((D, D))
    h1 = _tc_call(
        _tc_pre_a_kernel, [_spec_rows, sw], _spec_rows,
        jax.ShapeDtypeStruct((NP, D), jnp.float32), x_pad, W1)
    dinv, t1 = _tc_call(
        _tc_pre_b_kernel,
        [_spec_deg, _spec_rows],
        (_spec_rows1, _spec_t),
        (jax.ShapeDtypeStruct((NP, 1), jnp.float32), t_shape),
        deg, h1)

    mid_in = [_spec_p, _spec_t, _spec_rows1, sb, sw]
    p1 = _sc_spmm(t1, src, dst, zeros_rh)
    t2 = _tc_call(_tc_mid_kernel, mid_in, _spec_t, t_shape,
                  p1, t1, dinv, b1.reshape(1, D), W2)

    p2 = _sc_spmm(t2, src, dst, zeros_rh)
    t3 = _tc_call(_tc_mid_kernel, mid_in, _spec_t, t_shape,
                  p2, t2, dinv, b2.reshape(1, D), W3)

    p3 = _sc_spmm(t3, src, dst, zeros_rh)
    swih = _spec_full((4 * D, D))
    sbl = _spec_full((1, 4 * D))
    out = _tc_call(
        _tc_post_kernel,
        [_spec_p, _spec_t, _spec_rows1, sb,
         swih, sbl, swih, sbl, swih, sbl, sw, sb],
        _spec_rows,
        jax.ShapeDtypeStruct((NP, D), jnp.float32),
        p3, t3, dinv, b3.reshape(1, D),
        Wih0, (bih0 + bhh0).reshape(1, 4 * D),
        Wih1, (bih1 + bhh1).reshape(1, 4 * D),
        Wih2, (bih2 + bhh2).reshape(1, 4 * D),
        Wlin, blin.reshape(1, D))

    return out[:N]
